# baseline (speedup 1.0000x reference)
import functools

import jax
import jax.numpy as jnp
from jax import lax
from jax.experimental import pallas as pl
from jax.experimental.pallas import tpu as pltpu
from jax.experimental.pallas import tpu_sc as plsc

_N = 1000000
_B = 16384
_DIH_EPS = 0.076
_GNN_EPS = 6.38
_FS_EPS = 3.0
_LR = 0.001

_NS = 16
_PB = _B // _NS
_HALF = _N // 2
_QTR = _HALF // 2
_DUMMY = _QTR
_SH = _QTR + 8
_SCH = 15616
_TL = _QTR - _NS * _SCH


def _sc_body(idx_hbm, dih_hbm, gnn_hbm, fs_hbm, lamd_hbm, lamg_hbm, lamf_hbm,
             outd_hbm, outg_hbm, outf_hbm, part_hbm,
             sh0, sh1, sh2, bnc0, bnc1, bnc2, idx_v, lidx_v,
             loss0, loss1, loss2, lam0, lam1, lam2,
             new0, new1, new2, pacc, tailb,
             sem_st, sem_g, sem_sc, sem_out):
  cid = lax.axis_index("c")
  sid = lax.axis_index("s")
  shs = (sh0, sh1, sh2)
  bounce = (bnc0, bnc1, bnc2)
  losses = (loss0, loss1, loss2)
  lams = (lam0, lam1, lam2)
  news = (new0, new1, new2)
  bufs = ((dih_hbm, lamd_hbm, outd_hbm, _DIH_EPS),
          (gnn_hbm, lamg_hbm, outg_hbm, _GNN_EPS),
          (fs_hbm, lamf_hbm, outf_hbm, _FS_EPS))

  pltpu.sync_copy(idx_hbm.at[pl.ds(sid * _PB, _PB)], idx_v)
  for b, (loss_hbm, _, _, _) in enumerate(bufs):
    pltpu.sync_copy(loss_hbm.at[pl.ds(sid * _PB, _PB)], losses[b])

  for rnd in range(2):
    rbase = pl.multiple_of(cid * _HALF + rnd * _QTR, 8)

    def lidx_step(k, _):
      o = pl.multiple_of(k * 16, 16)
      idx = idx_v[pl.ds(o, 16)]
      owned = (idx >= rbase) & (idx < rbase + _QTR)
      lidx_v[pl.ds(o, 16)] = jnp.where(owned, idx - rbase, _DUMMY)
      return 0

    lax.fori_loop(0, _PB // 16, lidx_step, 0)

    off = sid * _SCH
    sts = [pltpu.async_copy(lam_hbm.at[pl.ds(rbase + off, _SCH)],
                            bounce[b], sem_st)
           for b, (_, lam_hbm, _, _) in enumerate(bufs)]
    for cp in sts:
      cp.wait()
    sts = [pltpu.async_copy(bounce[b], sh.at[pl.ds(off, _SCH)], sem_st)
           for b, sh in enumerate(shs)]
    for cp in sts:
      cp.wait()

    @pl.when(sid == 0)
    def _():
      for (_, lam_hbm, _, _), sh in zip(bufs, shs):
        pltpu.sync_copy(lam_hbm.at[pl.ds(rbase + _NS * _SCH, _TL)], tailb)
        pltpu.sync_copy(tailb, sh.at[pl.ds(_NS * _SCH, _TL)])

    plsc.subcore_barrier()

    gth = [pltpu.async_copy(sh.at[lidx_v], lam_v, sem_g)
           for sh, lam_v in zip(shs, lams)]
    for cp in gth:
      cp.wait()
    for b, (_, _, _, eps) in enumerate(bufs):
      loss_v, lam_v, new_v = losses[b], lams[b], news[b]

      def step(k, acc):
        o = pl.multiple_of(k * 16, 16)
        idx = idx_v[pl.ds(o, 16)]
        owned = (idx >= rbase) & (idx < rbase + _QTR)
        lam = lam_v[pl.ds(o, 16)]
        viol = loss_v[pl.ds(o, 16)] - eps
        new_v[pl.ds(o, 16)] = jnp.maximum(lam + _LR * viol, 0.0)
        return acc + jnp.where(owned, lam * viol, 0.0)

      acc = lax.fori_loop(0, _PB // 16, step, jnp.zeros((16,), jnp.float32))
      if rnd == 0:
        pacc[b, pl.ds(0, 16)] = acc
      else:
        pacc[b, pl.ds(0, 16)] = pacc[b, pl.ds(0, 16)] + acc

    plsc.subcore_barrier()

    scs = [pltpu.async_copy(new_v, sh.at[lidx_v], sem_sc)
           for sh, new_v in zip(shs, news)]
    for cp in scs:
      cp.wait()

    plsc.subcore_barrier()

    wos = [pltpu.async_copy(sh.at[pl.ds(off, _SCH)], bounce[b], sem_out)
           for b, sh in enumerate(shs)]
    for cp in wos:
      cp.wait()
    wos = [pltpu.async_copy(bounce[b], out_hbm.at[pl.ds(rbase + off, _SCH)],
                            sem_out)
           for b, (_, _, out_hbm, _) in enumerate(bufs)]
    for cp in wos:
      cp.wait()

    @pl.when(sid == 0)
    def _():
      for (_, _, out_hbm, _), sh in zip(bufs, shs):
        pltpu.sync_copy(sh.at[pl.ds(_NS * _SCH, _TL)], tailb)
        pltpu.sync_copy(tailb, out_hbm.at[pl.ds(rbase + _NS * _SCH, _TL)])

    plsc.subcore_barrier()

  pltpu.sync_copy(pacc, part_hbm.at[cid, sid])


_sc_call = functools.partial(
    pl.kernel,
    out_type=(
        jax.ShapeDtypeStruct((_N,), jnp.float32),
        jax.ShapeDtypeStruct((_N,), jnp.float32),
        jax.ShapeDtypeStruct((_N,), jnp.float32),
        jax.ShapeDtypeStruct((2, _NS, 3, 16), jnp.float32),
    ),
    mesh=plsc.VectorSubcoreMesh(core_axis_name="c", subcore_axis_name="s"),
    scratch_types=[
        pltpu.VMEM_SHARED((_SH,), jnp.float32),
        pltpu.VMEM_SHARED((_SH,), jnp.float32),
        pltpu.VMEM_SHARED((_SH,), jnp.float32),
        pltpu.VMEM((_SCH,), jnp.float32),
        pltpu.VMEM((_SCH,), jnp.float32),
        pltpu.VMEM((_SCH,), jnp.float32),
        pltpu.VMEM((_PB,), jnp.int32),
        pltpu.VMEM((_PB,), jnp.int32),
        pltpu.VMEM((_PB,), jnp.float32),
        pltpu.VMEM((_PB,), jnp.float32),
        pltpu.VMEM((_PB,), jnp.float32),
        pltpu.VMEM((_PB,), jnp.float32),
        pltpu.VMEM((_PB,), jnp.float32),
        pltpu.VMEM((_PB,), jnp.float32),
        pltpu.VMEM((_PB,), jnp.float32),
        pltpu.VMEM((_PB,), jnp.float32),
        pltpu.VMEM((_PB,), jnp.float32),
        pltpu.VMEM((3, 16), jnp.float32),
        pltpu.VMEM((_TL,), jnp.float32),
        pltpu.SemaphoreType.DMA,
        pltpu.SemaphoreType.DMA,
        pltpu.SemaphoreType.DMA,
        pltpu.SemaphoreType.DMA,
    ],
)(_sc_body)


def kernel(primary_loss, dihedral_losses, gnn_losses, foldseek_losses, indices,
           lam_dihedral, lam_gnn, lam_foldseek):
  out_d, out_g, out_f, part = _sc_call(
      indices.astype(jnp.int32), dihedral_losses, gnn_losses, foldseek_losses,
      lam_dihedral, lam_gnn, lam_foldseek)
  lagrangian = primary_loss + jnp.sum(part) / jnp.float32(_B)
  return lagrangian, out_d, out_g, out_f
